# Initial kernel scaffold; baseline (speedup 1.0000x reference)
import jax
import jax.numpy as jnp
from jax.experimental import pallas as pl


def kernel(x, edge_index, W1, b1, Wc, bc):
    raise NotImplementedError("write your pallas kernel here")


# speedup vs baseline: 38.6414x; 38.6414x over previous
import functools

import jax
import jax.numpy as jnp
from jax import lax
from jax.experimental import pallas as pl
from jax.experimental.pallas import tpu as pltpu
from jax.experimental.pallas import tpu_sc as plsc

N = 10000
NPAD = 10240
E = 320000
NFEAT = 128
NHID = 64
NCLASS = 16

NUM_CORES = 2
NUM_SUBCORES = 16
NUM_TILES = NUM_CORES * NUM_SUBCORES

CHUNK = 128
CHUNKS_PER_TILE = 80
E_PAD = NUM_TILES * CHUNKS_PER_TILE * CHUNK
NCHUNKS = E_PAD // CHUNK
ROWS_PER_SUBCORE = NPAD // NUM_SUBCORES

_BLK = 1024
_NBLK = 10

_f32 = jnp.float32


_sc_mesh = plsc.VectorSubcoreMesh(core_axis_name="c", subcore_axis_name="s")


@functools.partial(
    pl.kernel,
    mesh=_sc_mesh,
    out_type=jax.ShapeDtypeStruct((NUM_CORES, NPAD), _f32),
    compiler_params=pltpu.CompilerParams(use_tc_tiling_on_sc=False),
    scratch_types=[
        pltpu.VMEM((CHUNKS_PER_TILE, CHUNK), jnp.int32),
        pltpu.VMEM((CHUNK,), _f32),
        pltpu.VMEM((ROWS_PER_SUBCORE,), _f32),
        pltpu.VMEM_SHARED((NPAD,), _f32),
    ],
)
def _sc_hist(dst_hbm, out_hbm, dst_v, ones_v, zero_v, acc_sh):
    c = lax.axis_index("c")
    s = lax.axis_index("s")
    wid = s * NUM_CORES + c

    ones16 = jnp.ones((16,), _f32)
    zeros16 = jnp.zeros((16,), _f32)
    for i in range(CHUNK // 16):
        ones_v[pl.ds(i * 16, 16)] = ones16
    for i in range(ROWS_PER_SUBCORE // 16):
        zero_v[pl.ds(i * 16, 16)] = zeros16

    pltpu.sync_copy(zero_v, acc_sh.at[pl.ds(s * ROWS_PER_SUBCORE, ROWS_PER_SUBCORE)])
    plsc.subcore_barrier()

    pltpu.sync_copy(dst_hbm.at[pl.ds(wid * CHUNKS_PER_TILE, CHUNKS_PER_TILE)], dst_v)

    def body(j, carry):
        pltpu.sync_copy(ones_v, acc_sh.at[dst_v.at[j]], add=True)
        return carry

    lax.fori_loop(0, CHUNKS_PER_TILE, body, 0)
    plsc.subcore_barrier()

    pltpu.sync_copy(
        acc_sh.at[pl.ds(s * ROWS_PER_SUBCORE, ROWS_PER_SUBCORE)],
        out_hbm.at[c, pl.ds(s * ROWS_PER_SUBCORE, ROWS_PER_SUBCORE)],
    )


@functools.partial(
    pl.kernel,
    mesh=_sc_mesh,
    out_type=jax.ShapeDtypeStruct((NUM_CORES, NPAD, NHID), _f32),
    compiler_params=pltpu.CompilerParams(use_tc_tiling_on_sc=False),
    scratch_types=[
        pltpu.VMEM((CHUNKS_PER_TILE, CHUNK), jnp.int32),
        pltpu.VMEM((CHUNKS_PER_TILE, CHUNK), jnp.int32),
        pltpu.VMEM((CHUNK, NHID), _f32),
        pltpu.VMEM_SHARED((NPAD, NHID), _f32),
        pltpu.SemaphoreType.DMA,
    ],
)
def _sc_scatter(g_hbm, src_hbm, dst_hbm, out_hbm, src_v, dst_v, rows_v, acc_sh, sem):
    c = lax.axis_index("c")
    s = lax.axis_index("s")
    wid = s * NUM_CORES + c

    zeros16 = jnp.zeros((16,), _f32)
    for r in range(CHUNK):
        for k in range(NHID // 16):
            rows_v[r, pl.ds(k * 16, 16)] = zeros16
    for k in range(ROWS_PER_SUBCORE // CHUNK):
        pltpu.sync_copy(
            rows_v, acc_sh.at[pl.ds(s * ROWS_PER_SUBCORE + k * CHUNK, CHUNK)]
        )
    plsc.subcore_barrier()

    pltpu.sync_copy(src_hbm.at[pl.ds(wid * CHUNKS_PER_TILE, CHUNKS_PER_TILE)], src_v)
    pltpu.sync_copy(dst_hbm.at[pl.ds(wid * CHUNKS_PER_TILE, CHUNKS_PER_TILE)], dst_v)

    def body(j, carry):
        pltpu.async_copy(g_hbm.at[src_v.at[j]], rows_v, sem).wait()
        pltpu.sync_copy(rows_v, acc_sh.at[dst_v.at[j]], add=True)
        return carry

    lax.fori_loop(0, CHUNKS_PER_TILE, body, 0)
    plsc.subcore_barrier()

    pltpu.sync_copy(
        acc_sh.at[pl.ds(s * ROWS_PER_SUBCORE, ROWS_PER_SUBCORE)],
        out_hbm.at[c, pl.ds(s * ROWS_PER_SUBCORE, ROWS_PER_SUBCORE)],
    )


def _tc_a_body(x_ref, w_ref, d_ref, g_ref):
    deg = d_ref[0, :] + d_ref[1, :] + 1.0
    dinv = lax.rsqrt(deg)
    h = jnp.dot(x_ref[...], w_ref[...], preferred_element_type=_f32)
    g_ref[...] = h * dinv[:, None]


def _tc_a(x, W1, degp3):
    return pl.pallas_call(
        _tc_a_body,
        grid=(_NBLK,),
        in_specs=[
            pl.BlockSpec((_BLK, NFEAT), lambda i: (i, 0)),
            pl.BlockSpec((NFEAT, NHID), lambda i: (0, 0)),
            pl.BlockSpec((2, _BLK), lambda i: (0, i)),
        ],
        out_specs=pl.BlockSpec((_BLK, NHID), lambda i: (i, 0)),
        out_shape=jax.ShapeDtypeStruct((N, NHID), _f32),
    )(x, W1, degp3)


def _tc_b_body(s_ref, g_ref, d_ref, b1_ref, wc_ref, bc_ref, o_ref):
    deg = d_ref[0, :] + d_ref[1, :] + 1.0
    dinv = lax.rsqrt(deg)
    tot = s_ref[0] + s_ref[1] + g_ref[...]
    pre = tot * dinv[:, None] + b1_ref[...]
    h2 = jnp.maximum(pre, 0.0)
    logits = jnp.dot(h2, wc_ref[...], preferred_element_type=_f32) + bc_ref[...]
    m = jnp.max(logits, axis=1, keepdims=True)
    z = logits - m
    lse = jnp.log(jnp.sum(jnp.exp(z), axis=1, keepdims=True))
    o_ref[...] = z - lse


def _tc_b(S, g, degp3, b1, Wc, bc):
    return pl.pallas_call(
        _tc_b_body,
        grid=(_NBLK,),
        in_specs=[
            pl.BlockSpec((2, _BLK, NHID), lambda i: (0, i, 0)),
            pl.BlockSpec((_BLK, NHID), lambda i: (i, 0)),
            pl.BlockSpec((2, _BLK), lambda i: (0, i)),
            pl.BlockSpec((1, NHID), lambda i: (0, 0)),
            pl.BlockSpec((NHID, NCLASS), lambda i: (0, 0)),
            pl.BlockSpec((1, NCLASS), lambda i: (0, 0)),
        ],
        out_specs=pl.BlockSpec((_BLK, NCLASS), lambda i: (i, 0)),
        out_shape=jax.ShapeDtypeStruct((N, NCLASS), _f32),
    )(S, g, degp3, b1, Wc, bc)


def kernel(x, edge_index, W1, b1, Wc, bc):
    src = edge_index[0]
    dst = edge_index[1]
    npad_e = E_PAD - E
    ar = jnp.arange(npad_e, dtype=jnp.int32)
    src_p = jnp.concatenate([src, ar % N]).reshape(NCHUNKS, CHUNK)
    dst_p = jnp.concatenate([dst, N + ar % (NPAD - N)]).reshape(NCHUNKS, CHUNK)

    degp = _sc_hist(dst_p)
    g = _tc_a(x, W1, degp)
    S = _sc_scatter(g, src_p, dst_p)
    return _tc_b(S, g, degp, b1.reshape(1, NHID), Wc, bc.reshape(1, NCLASS))


# speedup vs baseline: 50.8786x; 1.3167x over previous
import functools

import jax
import jax.numpy as jnp
from jax import lax
from jax.experimental import pallas as pl
from jax.experimental.pallas import tpu as pltpu
from jax.experimental.pallas import tpu_sc as plsc

N = 10000
NPAD = 10240
E = 320000
NFEAT = 128
NHID = 64
NCLASS = 16

NUM_CORES = 2
NUM_SUBCORES = 16
NUM_TILES = NUM_CORES * NUM_SUBCORES

CHUNK = 128
CHUNKS_PER_TILE = 80
E_PAD = NUM_TILES * CHUNKS_PER_TILE * CHUNK
NCHUNKS = E_PAD // CHUNK
ROWS_PER_SUBCORE = NPAD // NUM_SUBCORES

_BLK = 1024
_NBLK = 10

_f32 = jnp.float32


_sc_mesh = plsc.VectorSubcoreMesh(core_axis_name="c", subcore_axis_name="s")


@functools.partial(
    pl.kernel,
    mesh=_sc_mesh,
    out_type=jax.ShapeDtypeStruct((NUM_CORES, NPAD), _f32),
    compiler_params=pltpu.CompilerParams(use_tc_tiling_on_sc=False),
    scratch_types=[
        pltpu.VMEM((CHUNKS_PER_TILE, CHUNK), jnp.int32),
        pltpu.VMEM((CHUNK,), _f32),
        pltpu.VMEM((ROWS_PER_SUBCORE,), _f32),
        pltpu.VMEM_SHARED((NPAD,), _f32),
    ],
)
def _sc_hist(dst_hbm, out_hbm, dst_v, ones_v, zero_v, acc_sh):
    c = lax.axis_index("c")
    s = lax.axis_index("s")
    wid = s * NUM_CORES + c

    ones16 = jnp.ones((16,), _f32)
    zeros16 = jnp.zeros((16,), _f32)
    for i in range(CHUNK // 16):
        ones_v[pl.ds(i * 16, 16)] = ones16
    for i in range(ROWS_PER_SUBCORE // 16):
        zero_v[pl.ds(i * 16, 16)] = zeros16

    pltpu.sync_copy(zero_v, acc_sh.at[pl.ds(s * ROWS_PER_SUBCORE, ROWS_PER_SUBCORE)])
    plsc.subcore_barrier()

    pltpu.sync_copy(dst_hbm.at[pl.ds(wid * CHUNKS_PER_TILE, CHUNKS_PER_TILE)], dst_v)

    def body(j, carry):
        pltpu.sync_copy(ones_v, acc_sh.at[dst_v.at[j]], add=True)
        return carry

    lax.fori_loop(0, CHUNKS_PER_TILE, body, 0)
    plsc.subcore_barrier()

    pltpu.sync_copy(
        acc_sh.at[pl.ds(s * ROWS_PER_SUBCORE, ROWS_PER_SUBCORE)],
        out_hbm.at[c, pl.ds(s * ROWS_PER_SUBCORE, ROWS_PER_SUBCORE)],
    )


@functools.partial(
    pl.kernel,
    mesh=_sc_mesh,
    out_type=jax.ShapeDtypeStruct((NUM_CORES, NPAD, NHID), _f32),
    compiler_params=pltpu.CompilerParams(use_tc_tiling_on_sc=False),
    scratch_types=[
        pltpu.VMEM((CHUNKS_PER_TILE, CHUNK), jnp.int32),
        pltpu.VMEM((CHUNKS_PER_TILE, CHUNK), jnp.int32),
        pltpu.VMEM((CHUNK, NHID), _f32),
        pltpu.VMEM((CHUNK, NHID), _f32),
        pltpu.VMEM_SHARED((NPAD, NHID), _f32),
        pltpu.SemaphoreType.DMA,
        pltpu.SemaphoreType.DMA,
    ],
)
def _sc_scatter(
    g_hbm, src_hbm, dst_hbm, out_hbm, src_v, dst_v, rows_a, rows_b, acc_sh, sem_a, sem_b
):
    c = lax.axis_index("c")
    s = lax.axis_index("s")
    wid = s * NUM_CORES + c

    zeros16 = jnp.zeros((16,), _f32)
    for r in range(CHUNK):
        for k in range(NHID // 16):
            rows_a[r, pl.ds(k * 16, 16)] = zeros16
    for k in range(ROWS_PER_SUBCORE // CHUNK):
        pltpu.sync_copy(
            rows_a, acc_sh.at[pl.ds(s * ROWS_PER_SUBCORE + k * CHUNK, CHUNK)]
        )
    plsc.subcore_barrier()

    pltpu.sync_copy(src_hbm.at[pl.ds(wid * CHUNKS_PER_TILE, CHUNKS_PER_TILE)], src_v)
    pltpu.sync_copy(dst_hbm.at[pl.ds(wid * CHUNKS_PER_TILE, CHUNKS_PER_TILE)], dst_v)

    pltpu.async_copy(g_hbm.at[src_v.at[0]], rows_a, sem_a)

    def body(t, carry):
        ja = 2 * t
        jb = ja + 1
        pltpu.async_copy(g_hbm.at[src_v.at[jb]], rows_b, sem_b)
        pltpu.make_async_copy(g_hbm.at[src_v.at[ja]], rows_a, sem_a).wait()
        pltpu.sync_copy(rows_a, acc_sh.at[dst_v.at[ja]], add=True)
        jn = jnp.minimum(ja + 2, CHUNKS_PER_TILE - 1)
        pltpu.async_copy(g_hbm.at[src_v.at[jn]], rows_a, sem_a)
        pltpu.make_async_copy(g_hbm.at[src_v.at[jb]], rows_b, sem_b).wait()
        pltpu.sync_copy(rows_b, acc_sh.at[dst_v.at[jb]], add=True)
        return carry

    lax.fori_loop(0, CHUNKS_PER_TILE // 2, body, 0)
    pltpu.make_async_copy(
        g_hbm.at[src_v.at[CHUNKS_PER_TILE - 1]], rows_a, sem_a
    ).wait()
    plsc.subcore_barrier()

    pltpu.sync_copy(
        acc_sh.at[pl.ds(s * ROWS_PER_SUBCORE, ROWS_PER_SUBCORE)],
        out_hbm.at[c, pl.ds(s * ROWS_PER_SUBCORE, ROWS_PER_SUBCORE)],
    )


def _tc_a_body(x_ref, w_ref, d_ref, g_ref):
    deg = d_ref[0, :] + d_ref[1, :] + 1.0
    dinv = lax.rsqrt(deg)
    h = jnp.dot(x_ref[...], w_ref[...], preferred_element_type=_f32)
    g_ref[...] = h * dinv[:, None]


def _tc_a(x, W1, degp3):
    return pl.pallas_call(
        _tc_a_body,
        grid=(_NBLK,),
        in_specs=[
            pl.BlockSpec((_BLK, NFEAT), lambda i: (i, 0)),
            pl.BlockSpec((NFEAT, NHID), lambda i: (0, 0)),
            pl.BlockSpec((2, _BLK), lambda i: (0, i)),
        ],
        out_specs=pl.BlockSpec((_BLK, NHID), lambda i: (i, 0)),
        out_shape=jax.ShapeDtypeStruct((N, NHID), _f32),
    )(x, W1, degp3)


def _tc_b_body(s_ref, g_ref, d_ref, b1_ref, wc_ref, bc_ref, o_ref):
    deg = d_ref[0, :] + d_ref[1, :] + 1.0
    dinv = lax.rsqrt(deg)
    tot = s_ref[0] + s_ref[1] + g_ref[...]
    pre = tot * dinv[:, None] + b1_ref[...]
    h2 = jnp.maximum(pre, 0.0)
    logits = jnp.dot(h2, wc_ref[...], preferred_element_type=_f32) + bc_ref[...]
    m = jnp.max(logits, axis=1, keepdims=True)
    z = logits - m
    lse = jnp.log(jnp.sum(jnp.exp(z), axis=1, keepdims=True))
    o_ref[...] = z - lse


def _tc_b(S, g, degp3, b1, Wc, bc):
    return pl.pallas_call(
        _tc_b_body,
        grid=(_NBLK,),
        in_specs=[
            pl.BlockSpec((2, _BLK, NHID), lambda i: (0, i, 0)),
            pl.BlockSpec((_BLK, NHID), lambda i: (i, 0)),
            pl.BlockSpec((2, _BLK), lambda i: (0, i)),
            pl.BlockSpec((1, NHID), lambda i: (0, 0)),
            pl.BlockSpec((NHID, NCLASS), lambda i: (0, 0)),
            pl.BlockSpec((1, NCLASS), lambda i: (0, 0)),
        ],
        out_specs=pl.BlockSpec((_BLK, NCLASS), lambda i: (i, 0)),
        out_shape=jax.ShapeDtypeStruct((N, NCLASS), _f32),
    )(S, g, degp3, b1, Wc, bc)


def kernel(x, edge_index, W1, b1, Wc, bc):
    src = edge_index[0]
    dst = edge_index[1]
    npad_e = E_PAD - E
    ar = jnp.arange(npad_e, dtype=jnp.int32)
    src_p = jnp.concatenate([src, ar % N]).reshape(NCHUNKS, CHUNK)
    dst_p = jnp.concatenate([dst, N + ar % (NPAD - N)]).reshape(NCHUNKS, CHUNK)

    degp = _sc_hist(dst_p)
    g = _tc_a(x, W1, degp)
    S = _sc_scatter(g, src_p, dst_p)
    return _tc_b(S, g, degp, b1.reshape(1, NHID), Wc, bc.reshape(1, NCLASS))


# speedup vs baseline: 53.2643x; 1.0469x over previous
import functools

import jax
import jax.numpy as jnp
from jax import lax
from jax.experimental import pallas as pl
from jax.experimental.pallas import tpu as pltpu
from jax.experimental.pallas import tpu_sc as plsc

N = 10000
NPAD = 10240
E = 320000
NFEAT = 128
NHID = 64
NCLASS = 16

NUM_CORES = 2
NUM_SUBCORES = 16
NUM_TILES = NUM_CORES * NUM_SUBCORES

CHUNK = 128
CHUNKS_PER_TILE = 80
E_PAD = NUM_TILES * CHUNKS_PER_TILE * CHUNK
NCHUNKS = E_PAD // CHUNK
ROWS_PER_SUBCORE = NPAD // NUM_SUBCORES

_BLK = 1024
_NBLK = 10

_f32 = jnp.float32


_sc_mesh = plsc.VectorSubcoreMesh(core_axis_name="c", subcore_axis_name="s")


@functools.partial(
    pl.kernel,
    mesh=_sc_mesh,
    out_type=jax.ShapeDtypeStruct((NUM_CORES, NPAD), _f32),
    compiler_params=pltpu.CompilerParams(use_tc_tiling_on_sc=False),
    scratch_types=[
        pltpu.VMEM((CHUNKS_PER_TILE, CHUNK), jnp.int32),
        pltpu.VMEM((CHUNK,), _f32),
        pltpu.VMEM((ROWS_PER_SUBCORE,), _f32),
        pltpu.VMEM_SHARED((NPAD,), _f32),
    ],
)
def _sc_hist(dst_hbm, out_hbm, dst_v, ones_v, zero_v, acc_sh):
    c = lax.axis_index("c")
    s = lax.axis_index("s")
    wid = s * NUM_CORES + c

    ones16 = jnp.ones((16,), _f32)
    zeros16 = jnp.zeros((16,), _f32)
    for i in range(CHUNK // 16):
        ones_v[pl.ds(i * 16, 16)] = ones16
    for i in range(ROWS_PER_SUBCORE // 16):
        zero_v[pl.ds(i * 16, 16)] = zeros16

    pltpu.sync_copy(zero_v, acc_sh.at[pl.ds(s * ROWS_PER_SUBCORE, ROWS_PER_SUBCORE)])
    plsc.subcore_barrier()

    pltpu.sync_copy(dst_hbm.at[pl.ds(wid * CHUNKS_PER_TILE, CHUNKS_PER_TILE)], dst_v)

    def body(j, carry):
        pltpu.sync_copy(ones_v, acc_sh.at[dst_v.at[j]], add=True)
        return carry

    lax.fori_loop(0, CHUNKS_PER_TILE, body, 0)
    plsc.subcore_barrier()

    pltpu.sync_copy(
        acc_sh.at[pl.ds(s * ROWS_PER_SUBCORE, ROWS_PER_SUBCORE)],
        out_hbm.at[c, pl.ds(s * ROWS_PER_SUBCORE, ROWS_PER_SUBCORE)],
    )


@functools.partial(
    pl.kernel,
    mesh=_sc_mesh,
    out_type=jax.ShapeDtypeStruct((NUM_CORES, NPAD, NHID), _f32),
    compiler_params=pltpu.CompilerParams(use_tc_tiling_on_sc=False),
    scratch_types=[
        pltpu.VMEM((CHUNKS_PER_TILE, CHUNK), jnp.int32),
        pltpu.VMEM((CHUNKS_PER_TILE, CHUNK), jnp.int32),
        pltpu.VMEM((4, CHUNK, NHID), _f32),
        pltpu.VMEM_SHARED((NPAD, NHID), _f32),
        pltpu.SemaphoreType.DMA,
        pltpu.SemaphoreType.DMA,
        pltpu.SemaphoreType.DMA,
        pltpu.SemaphoreType.DMA,
        pltpu.SemaphoreType.DMA,
        pltpu.SemaphoreType.DMA,
        pltpu.SemaphoreType.DMA,
        pltpu.SemaphoreType.DMA,
    ],
)
def _sc_scatter(
    g_hbm, src_hbm, dst_hbm, out_hbm, src_v, dst_v, rows_v,
    acc_sh, g0, g1, g2, g3, s0, s1, s2, s3
):
    c = lax.axis_index("c")
    s = lax.axis_index("s")
    wid = s * NUM_CORES + c
    gsem = (g0, g1, g2, g3)
    ssem = (s0, s1, s2, s3)

    zeros16 = jnp.zeros((16,), _f32)
    for r in range(CHUNK):
        for k in range(NHID // 16):
            rows_v[0, r, pl.ds(k * 16, 16)] = zeros16
    for k in range(ROWS_PER_SUBCORE // CHUNK):
        pltpu.sync_copy(
            rows_v.at[0], acc_sh.at[pl.ds(s * ROWS_PER_SUBCORE + k * CHUNK, CHUNK)]
        )
    plsc.subcore_barrier()

    pltpu.sync_copy(src_hbm.at[pl.ds(wid * CHUNKS_PER_TILE, CHUNKS_PER_TILE)], src_v)
    pltpu.sync_copy(dst_hbm.at[pl.ds(wid * CHUNKS_PER_TILE, CHUNKS_PER_TILE)], dst_v)

    for k in range(4):
        pltpu.async_copy(g_hbm.at[src_v.at[k]], rows_v.at[k], gsem[k])

    def body(t, carry):
        for k in range(4):
            j = 4 * t + k
            pltpu.make_async_copy(g_hbm.at[src_v.at[j]], rows_v.at[k], gsem[k]).wait()
            pltpu.async_copy(rows_v.at[k], acc_sh.at[dst_v.at[j]], ssem[k], add=True)
        for k in range(4):
            j = 4 * t + k
            jn = jnp.minimum(j + 4, CHUNKS_PER_TILE - 1)
            pltpu.make_async_copy(rows_v.at[k], acc_sh.at[dst_v.at[j]], ssem[k]).wait()
            pltpu.async_copy(g_hbm.at[src_v.at[jn]], rows_v.at[k], gsem[k])
        return carry

    lax.fori_loop(0, CHUNKS_PER_TILE // 4, body, 0)
    for k in range(4):
        pltpu.make_async_copy(
            g_hbm.at[src_v.at[CHUNKS_PER_TILE - 1]], rows_v.at[k], gsem[k]
        ).wait()
    plsc.subcore_barrier()

    pltpu.sync_copy(
        acc_sh.at[pl.ds(s * ROWS_PER_SUBCORE, ROWS_PER_SUBCORE)],
        out_hbm.at[c, pl.ds(s * ROWS_PER_SUBCORE, ROWS_PER_SUBCORE)],
    )


def _tc_a_body(x_ref, w_ref, d0_ref, d1_ref, g_ref):
    deg = d0_ref[...] + d1_ref[...] + 1.0
    dinv = lax.rsqrt(deg)
    h = jnp.dot(x_ref[...], w_ref[...], preferred_element_type=_f32)
    g_ref[...] = h * dinv[:, None]


def _tc_a(x, W1, degp_flat):
    return pl.pallas_call(
        _tc_a_body,
        grid=(_NBLK,),
        in_specs=[
            pl.BlockSpec((_BLK, NFEAT), lambda i: (i, 0)),
            pl.BlockSpec((NFEAT, NHID), lambda i: (0, 0)),
            pl.BlockSpec((_BLK,), lambda i: (i,)),
            pl.BlockSpec((_BLK,), lambda i: (i + _NBLK,)),
        ],
        out_specs=pl.BlockSpec((_BLK, NHID), lambda i: (i, 0)),
        out_shape=jax.ShapeDtypeStruct((N, NHID), _f32),
    )(x, W1, degp_flat, degp_flat)


def _tc_b_body(s_ref, g_ref, d0_ref, d1_ref, b1_ref, wc_ref, bc_ref, o_ref):
    deg = d0_ref[...] + d1_ref[...] + 1.0
    dinv = lax.rsqrt(deg)
    tot = s_ref[0] + s_ref[1] + g_ref[...]
    pre = tot * dinv[:, None] + b1_ref[...]
    h2 = jnp.maximum(pre, 0.0)
    logits = jnp.dot(h2, wc_ref[...], preferred_element_type=_f32) + bc_ref[...]
    m = jnp.max(logits, axis=1, keepdims=True)
    z = logits - m
    lse = jnp.log(jnp.sum(jnp.exp(z), axis=1, keepdims=True))
    o_ref[...] = z - lse


def _tc_b(S, g, degp_flat, b1, Wc, bc):
    return pl.pallas_call(
        _tc_b_body,
        grid=(_NBLK,),
        in_specs=[
            pl.BlockSpec((2, _BLK, NHID), lambda i: (0, i, 0)),
            pl.BlockSpec((_BLK, NHID), lambda i: (i, 0)),
            pl.BlockSpec((_BLK,), lambda i: (i,)),
            pl.BlockSpec((_BLK,), lambda i: (i + _NBLK,)),
            pl.BlockSpec((1, NHID), lambda i: (0, 0)),
            pl.BlockSpec((NHID, NCLASS), lambda i: (0, 0)),
            pl.BlockSpec((1, NCLASS), lambda i: (0, 0)),
        ],
        out_specs=pl.BlockSpec((_BLK, NCLASS), lambda i: (i, 0)),
        out_shape=jax.ShapeDtypeStruct((N, NCLASS), _f32),
    )(S, g, degp_flat, degp_flat, b1, Wc, bc)


def kernel(x, edge_index, W1, b1, Wc, bc):
    src = edge_index[0]
    dst = edge_index[1]
    npad_e = E_PAD - E
    ar = jnp.arange(npad_e, dtype=jnp.int32)
    src_p = jnp.concatenate([src, ar % N]).reshape(NCHUNKS, CHUNK)
    dst_p = jnp.concatenate([dst, N + ar % (NPAD - N)]).reshape(NCHUNKS, CHUNK)

    degp = _sc_hist(dst_p).reshape(-1)
    g = _tc_a(x, W1, degp)
    S = _sc_scatter(g, src_p, dst_p)
    return _tc_b(S, g, degp, b1.reshape(1, NHID), Wc, bc.reshape(1, NCLASS))


# speedup vs baseline: 58.9079x; 1.1060x over previous
import functools

import jax
import jax.numpy as jnp
from jax import lax
from jax.experimental import pallas as pl
from jax.experimental.pallas import tpu as pltpu
from jax.experimental.pallas import tpu_sc as plsc

N = 10000
NPAD = 10240
E = 320000
NFEAT = 128
NHID = 64
NCLASS = 16

NUM_CORES = 2
NUM_SUBCORES = 16
NUM_TILES = NUM_CORES * NUM_SUBCORES

CHUNK = 128
NCHUNKS = E // CHUNK
BASE_CHUNKS = NCHUNKS // NUM_TILES
EXTRA_BASE = BASE_CHUNKS * NUM_TILES
N_EXTRA = NCHUNKS - EXTRA_BASE
ROWS_PER_SUBCORE = NPAD // NUM_SUBCORES

_BLK = 1024
_NBLK = 10

_f32 = jnp.float32


_sc_mesh = plsc.VectorSubcoreMesh(core_axis_name="c", subcore_axis_name="s")


@functools.partial(
    pl.kernel,
    mesh=_sc_mesh,
    out_type=jax.ShapeDtypeStruct((NUM_CORES, NPAD), _f32),
    compiler_params=pltpu.CompilerParams(use_tc_tiling_on_sc=False),
    scratch_types=[
        pltpu.VMEM((BASE_CHUNKS + 1, 2, CHUNK), jnp.int32),
        pltpu.VMEM((CHUNK,), _f32),
        pltpu.VMEM((ROWS_PER_SUBCORE,), _f32),
        pltpu.VMEM_SHARED((NPAD,), _f32),
    ],
)
def _sc_hist(eil_hbm, out_hbm, eil_v, ones_v, zero_v, acc_sh):
    c = lax.axis_index("c")
    s = lax.axis_index("s")
    wid = s * NUM_CORES + c

    ones16 = jnp.ones((16,), _f32)
    zeros16 = jnp.zeros((16,), _f32)
    for i in range(CHUNK // 16):
        ones_v[pl.ds(i * 16, 16)] = ones16
    for i in range(ROWS_PER_SUBCORE // 16):
        zero_v[pl.ds(i * 16, 16)] = zeros16

    pltpu.sync_copy(zero_v, acc_sh.at[pl.ds(s * ROWS_PER_SUBCORE, ROWS_PER_SUBCORE)])
    plsc.subcore_barrier()

    pltpu.sync_copy(eil_hbm.at[pl.ds(wid * BASE_CHUNKS, BASE_CHUNKS)],
                    eil_v.at[pl.ds(0, BASE_CHUNKS)])

    @pl.when(wid < N_EXTRA)
    def _():
        pltpu.sync_copy(eil_hbm.at[pl.ds(EXTRA_BASE + wid, 1)],
                        eil_v.at[pl.ds(BASE_CHUNKS, 1)])

    def body(j, carry):
        pltpu.sync_copy(ones_v, acc_sh.at[eil_v.at[j, 1]], add=True)
        return carry

    lax.fori_loop(0, BASE_CHUNKS, body, 0)

    @pl.when(wid < N_EXTRA)
    def _():
        pltpu.sync_copy(ones_v, acc_sh.at[eil_v.at[BASE_CHUNKS, 1]], add=True)

    plsc.subcore_barrier()

    pltpu.sync_copy(
        acc_sh.at[pl.ds(s * ROWS_PER_SUBCORE, ROWS_PER_SUBCORE)],
        out_hbm.at[c, pl.ds(s * ROWS_PER_SUBCORE, ROWS_PER_SUBCORE)],
    )


_PIPE = BASE_CHUNKS // 4 * 4


@functools.partial(
    pl.kernel,
    mesh=_sc_mesh,
    out_type=jax.ShapeDtypeStruct((NUM_CORES, NPAD, NHID), _f32),
    compiler_params=pltpu.CompilerParams(use_tc_tiling_on_sc=False),
    scratch_types=[
        pltpu.VMEM((BASE_CHUNKS + 1, 2, CHUNK), jnp.int32),
        pltpu.VMEM((4, CHUNK, NHID), _f32),
        pltpu.VMEM_SHARED((NPAD, NHID), _f32),
        pltpu.SemaphoreType.DMA,
        pltpu.SemaphoreType.DMA,
        pltpu.SemaphoreType.DMA,
        pltpu.SemaphoreType.DMA,
        pltpu.SemaphoreType.DMA,
        pltpu.SemaphoreType.DMA,
        pltpu.SemaphoreType.DMA,
        pltpu.SemaphoreType.DMA,
    ],
)
def _sc_scatter(
    g_hbm, eil_hbm, out_hbm, eil_v, rows_v,
    acc_sh, g0, g1, g2, g3, s0, s1, s2, s3
):
    c = lax.axis_index("c")
    s = lax.axis_index("s")
    wid = s * NUM_CORES + c
    gsem = (g0, g1, g2, g3)
    ssem = (s0, s1, s2, s3)

    zeros16 = jnp.zeros((16,), _f32)
    for r in range(CHUNK):
        for k in range(NHID // 16):
            rows_v[0, r, pl.ds(k * 16, 16)] = zeros16
    for k in range(ROWS_PER_SUBCORE // CHUNK):
        pltpu.sync_copy(
            rows_v.at[0], acc_sh.at[pl.ds(s * ROWS_PER_SUBCORE + k * CHUNK, CHUNK)]
        )
    plsc.subcore_barrier()

    pltpu.sync_copy(eil_hbm.at[pl.ds(wid * BASE_CHUNKS, BASE_CHUNKS)],
                    eil_v.at[pl.ds(0, BASE_CHUNKS)])

    @pl.when(wid < N_EXTRA)
    def _():
        pltpu.sync_copy(eil_hbm.at[pl.ds(EXTRA_BASE + wid, 1)],
                        eil_v.at[pl.ds(BASE_CHUNKS, 1)])

    for k in range(4):
        pltpu.async_copy(g_hbm.at[eil_v.at[k, 0]], rows_v.at[k], gsem[k])

    def body(t, carry):
        for k in range(4):
            j = 4 * t + k
            pltpu.make_async_copy(g_hbm.at[eil_v.at[j, 0]], rows_v.at[k], gsem[k]).wait()
            pltpu.async_copy(rows_v.at[k], acc_sh.at[eil_v.at[j, 1]], ssem[k], add=True)
        for k in range(4):
            j = 4 * t + k
            jn = jnp.minimum(j + 4, _PIPE + 1)
            pltpu.make_async_copy(rows_v.at[k], acc_sh.at[eil_v.at[j, 1]], ssem[k]).wait()
            pltpu.async_copy(g_hbm.at[eil_v.at[jn, 0]], rows_v.at[k], gsem[k])
        return carry

    lax.fori_loop(0, _PIPE // 4, body, 0)
    for k in range(4):
        pltpu.make_async_copy(
            g_hbm.at[eil_v.at[_PIPE + (k if k < 2 else 1), 0]], rows_v.at[k], gsem[k]
        ).wait()
    for k in range(2):
        pltpu.sync_copy(rows_v.at[k], acc_sh.at[eil_v.at[_PIPE + k, 1]], add=True)

    @pl.when(wid < N_EXTRA)
    def _():
        pltpu.async_copy(
            g_hbm.at[eil_v.at[BASE_CHUNKS, 0]], rows_v.at[2], g2
        ).wait()
        pltpu.sync_copy(rows_v.at[2], acc_sh.at[eil_v.at[BASE_CHUNKS, 1]], add=True)

    plsc.subcore_barrier()

    pltpu.sync_copy(
        acc_sh.at[pl.ds(s * ROWS_PER_SUBCORE, ROWS_PER_SUBCORE)],
        out_hbm.at[c, pl.ds(s * ROWS_PER_SUBCORE, ROWS_PER_SUBCORE)],
    )


def _tc_a_body(x_ref, w_ref, d0_ref, d1_ref, g_ref):
    deg = d0_ref[...] + d1_ref[...] + 1.0
    dinv = lax.rsqrt(deg)
    h = jnp.dot(x_ref[...], w_ref[...], preferred_element_type=_f32)
    g_ref[...] = h * dinv[:, None]


def _tc_a(x, W1, degp_flat):
    return pl.pallas_call(
        _tc_a_body,
        grid=(_NBLK,),
        in_specs=[
            pl.BlockSpec((_BLK, NFEAT), lambda i: (i, 0)),
            pl.BlockSpec((NFEAT, NHID), lambda i: (0, 0)),
            pl.BlockSpec((_BLK,), lambda i: (i,)),
            pl.BlockSpec((_BLK,), lambda i: (i + _NBLK,)),
        ],
        out_specs=pl.BlockSpec((_BLK, NHID), lambda i: (i, 0)),
        out_shape=jax.ShapeDtypeStruct((N, NHID), _f32),
    )(x, W1, degp_flat, degp_flat)


def _tc_b_body(s_ref, g_ref, d0_ref, d1_ref, b1_ref, wc_ref, bc_ref, o_ref):
    deg = d0_ref[...] + d1_ref[...] + 1.0
    dinv = lax.rsqrt(deg)
    tot = s_ref[0] + s_ref[1] + g_ref[...]
    pre = tot * dinv[:, None] + b1_ref[...]
    h2 = jnp.maximum(pre, 0.0)
    logits = jnp.dot(h2, wc_ref[...], preferred_element_type=_f32) + bc_ref[...]
    m = jnp.max(logits, axis=1, keepdims=True)
    z = logits - m
    lse = jnp.log(jnp.sum(jnp.exp(z), axis=1, keepdims=True))
    o_ref[...] = z - lse


def _tc_b(S, g, degp_flat, b1, Wc, bc):
    return pl.pallas_call(
        _tc_b_body,
        grid=(_NBLK,),
        in_specs=[
            pl.BlockSpec((2, _BLK, NHID), lambda i: (0, i, 0)),
            pl.BlockSpec((_BLK, NHID), lambda i: (i, 0)),
            pl.BlockSpec((_BLK,), lambda i: (i,)),
            pl.BlockSpec((_BLK,), lambda i: (i + _NBLK,)),
            pl.BlockSpec((1, NHID), lambda i: (0, 0)),
            pl.BlockSpec((NHID, NCLASS), lambda i: (0, 0)),
            pl.BlockSpec((1, NCLASS), lambda i: (0, 0)),
        ],
        out_specs=pl.BlockSpec((_BLK, NCLASS), lambda i: (i, 0)),
        out_shape=jax.ShapeDtypeStruct((N, NCLASS), _f32),
    )(S, g, degp_flat, degp_flat, b1, Wc, bc)


def kernel(x, edge_index, W1, b1, Wc, bc):
    eil = edge_index.reshape(2, NCHUNKS, CHUNK).transpose(1, 0, 2)

    degp = _sc_hist(eil).reshape(-1)
    g = _tc_a(x, W1, degp)
    S = _sc_scatter(g, eil)
    return _tc_b(S, g, degp, b1.reshape(1, NHID), Wc, bc.reshape(1, NCLASS))
